# baseline (speedup 1.0000x reference)
import functools
import math

import jax
import jax.numpy as jnp
from jax import lax
from jax.experimental import pallas as pl
from jax.experimental.pallas import tpu as pltpu
from jax.experimental.pallas import tpu_sc as plsc

D_MODEL = 64
D_PAD = 128
SCALE = math.sqrt(D_MODEL)
VOCAB = 100000

NC, NS = 2, 16
NW = NC * NS

B_TOKENS = 4096 * 200
BPW = B_TOKENS // NW
SUB = 128
K = 2
CHUNK = SUB * K
NCHUNK = BPW // CHUNK
NROWS_W = BPW // SUB


def _scale_body(lut_ref, out_ref):
    v = lut_ref[...] * SCALE
    out_ref[...] = jnp.concatenate([v, jnp.zeros_like(v)], axis=1)


def _scaled_table(lut):
    return pl.pallas_call(
        _scale_body,
        out_shape=jax.ShapeDtypeStruct((VOCAB, D_PAD), jnp.float32),
        grid=(10,),
        in_specs=[pl.BlockSpec((VOCAB // 10, D_MODEL), lambda i: (i, 0))],
        out_specs=pl.BlockSpec((VOCAB // 10, D_PAD), lambda i: (i, 0)),
    )(lut)


NSEQ, TSEQ = 4096, 200
SEQ_W = NSEQ // NW
SPC = 2
CHUNK_T = SPC * TSEQ
NCHUNK_S = SEQ_W // SPC


@functools.partial(
    pl.kernel,
    out_type=jax.ShapeDtypeStruct((B_TOKENS, D_PAD), jnp.float32),
    mesh=plsc.VectorSubcoreMesh(core_axis_name="c", subcore_axis_name="s"),
    scratch_types=[
        pltpu.VMEM((SEQ_W, TSEQ), jnp.int32),
        pltpu.VMEM((2, CHUNK_T, D_PAD), jnp.float32),
        pltpu.SemaphoreType.DMA,
        pltpu.SemaphoreType.DMA,
        pltpu.SemaphoreType.DMA,
        pltpu.SemaphoreType.DMA,
    ],
    compiler_params=pltpu.CompilerParams(use_tc_tiling_on_sc=False),
)
def _sc_gather(idx_hbm, tab_hbm, out_hbm, idx_all, rows_v,
               sem_g0, sem_g1, sem_o0, sem_o1):
    wid = lax.axis_index("s") * NC + lax.axis_index("c")
    seq_base = wid * SEQ_W
    sem_g = (sem_g0, sem_g1)
    sem_o = (sem_o0, sem_o1)

    pltpu.sync_copy(idx_hbm.at[pl.ds(seq_base, SEQ_W)], idx_all)

    def fire_gathers(ii, b):
        for s in range(SPC):
            seq = ii * SPC + s
            pltpu.async_copy(
                tab_hbm.at[idx_all.at[seq, pl.ds(0, 128)]],
                rows_v.at[b].at[pl.ds(s * TSEQ, 128)],
                sem_g[b],
            )
            pltpu.async_copy(
                tab_hbm.at[idx_all.at[seq, pl.ds(128, TSEQ - 128)]],
                rows_v.at[b].at[pl.ds(s * TSEQ + 128, TSEQ - 128)],
                sem_g[b],
            )

    def out_slice(ii):
        return out_hbm.at[pl.ds((seq_base + ii * SPC) * TSEQ, CHUNK_T)]

    fire_gathers(0, 0)

    @pl.loop(0, NCHUNK_S, step=2)
    def _chunk(i):
        for b in range(2):
            ii = i + b
            b2 = 1 - b

            @pl.when(jnp.logical_and(ii >= 1, ii + 1 < NCHUNK_S))
            def _drain_out():
                pltpu.make_async_copy(rows_v.at[b2], out_slice(ii), sem_o[b2]).wait()

            @pl.when(ii + 1 < NCHUNK_S)
            def _next_gathers():
                fire_gathers(ii + 1, b2)

            pltpu.make_async_copy(tab_hbm.at[pl.ds(0, CHUNK_T)], rows_v.at[b], sem_g[b]).wait()
            pltpu.async_copy(rows_v.at[b], out_slice(ii), sem_o[b])

    for b in range(2):
        pltpu.make_async_copy(rows_v.at[b], out_slice(b), sem_o[b]).wait()


def kernel(x, lut):
    tab = _scaled_table(lut)
    out = _sc_gather(x.astype(jnp.int32), tab)
    return out[:, :D_MODEL].reshape(4096, 200, D_MODEL)


# speedup vs baseline: 1.0188x; 1.0188x over previous
import functools
import math

import jax
import jax.numpy as jnp
from jax import lax
from jax.experimental import pallas as pl
from jax.experimental.pallas import tpu as pltpu
from jax.experimental.pallas import tpu_sc as plsc

D_MODEL = 64
D_PAD = 128
SCALE = math.sqrt(D_MODEL)
VOCAB = 100000

NC, NS = 2, 16
NW = NC * NS

B_TOKENS = 4096 * 200
BPW = B_TOKENS // NW
SUB = 128
K = 2
CHUNK = SUB * K
NCHUNK = BPW // CHUNK
NROWS_W = BPW // SUB


def _scale_body(lut_ref, out_ref):
    v = lut_ref[...] * SCALE
    out_ref[...] = jnp.concatenate([v, jnp.zeros_like(v)], axis=1)


def _scaled_table(lut):
    return pl.pallas_call(
        _scale_body,
        out_shape=jax.ShapeDtypeStruct((VOCAB, D_PAD), jnp.float32),
        grid=(10,),
        in_specs=[pl.BlockSpec((VOCAB // 10, D_MODEL), lambda i: (i, 0))],
        out_specs=pl.BlockSpec((VOCAB // 10, D_PAD), lambda i: (i, 0)),
    )(lut)


NSEQ, TSEQ = 4096, 200
SEQ_W = NSEQ // NW
SPC = 1
CHUNK_T = SPC * TSEQ
NCHUNK_S = SEQ_W // SPC


@functools.partial(
    pl.kernel,
    out_type=jax.ShapeDtypeStruct((B_TOKENS, D_PAD), jnp.float32),
    mesh=plsc.VectorSubcoreMesh(core_axis_name="c", subcore_axis_name="s"),
    scratch_types=[
        pltpu.VMEM((SEQ_W, 128), jnp.int32),
        pltpu.VMEM((SEQ_W, 128), jnp.int32),
        pltpu.VMEM((2, CHUNK_T, D_PAD), jnp.float32),
        pltpu.SemaphoreType.DMA,
        pltpu.SemaphoreType.DMA,
        pltpu.SemaphoreType.DMA,
        pltpu.SemaphoreType.DMA,
    ],
    compiler_params=pltpu.CompilerParams(use_tc_tiling_on_sc=False),
)
def _sc_gather(xa_hbm, xb_hbm, tab_hbm, out_hbm, idx_a, idx_b, rows_v,
               sem_g0, sem_g1, sem_o0, sem_o1):
    wid = lax.axis_index("s") * NC + lax.axis_index("c")
    seq_base = wid * SEQ_W
    sem_g = (sem_g0, sem_g1)
    sem_o = (sem_o0, sem_o1)

    pltpu.sync_copy(xa_hbm.at[pl.ds(seq_base, SEQ_W)], idx_a)
    pltpu.sync_copy(xb_hbm.at[pl.ds(seq_base, SEQ_W)], idx_b)

    def fire_gathers(ii, b):
        for s in range(SPC):
            seq = ii * SPC + s
            pltpu.async_copy(
                tab_hbm.at[idx_a.at[seq]],
                rows_v.at[b].at[pl.ds(s * TSEQ, 128)],
                sem_g[b],
            )
            pltpu.async_copy(
                tab_hbm.at[idx_b.at[seq, pl.ds(0, TSEQ - 128)]],
                rows_v.at[b].at[pl.ds(s * TSEQ + 128, TSEQ - 128)],
                sem_g[b],
            )

    def out_slice(ii):
        return out_hbm.at[pl.ds((seq_base + ii * SPC) * TSEQ, CHUNK_T)]

    fire_gathers(0, 0)

    @pl.loop(0, NCHUNK_S, step=2)
    def _chunk(i):
        for b in range(2):
            ii = i + b
            b2 = 1 - b

            @pl.when(jnp.logical_and(ii >= 1, ii + 1 < NCHUNK_S))
            def _drain_out():
                pltpu.make_async_copy(rows_v.at[b2], out_slice(ii), sem_o[b2]).wait()

            @pl.when(ii + 1 < NCHUNK_S)
            def _next_gathers():
                fire_gathers(ii + 1, b2)

            pltpu.make_async_copy(tab_hbm.at[pl.ds(0, CHUNK_T)], rows_v.at[b], sem_g[b]).wait()
            pltpu.async_copy(rows_v.at[b], out_slice(ii), sem_o[b])

    for b in range(2):
        pltpu.make_async_copy(rows_v.at[b], out_slice(b), sem_o[b]).wait()


def kernel(x, lut):
    xi = x.astype(jnp.int32)
    xa = xi[:, :128]
    xb = jnp.pad(xi[:, 128:], ((0, 0), (0, 256 - TSEQ)))
    tab = _scaled_table(lut)
    out = _sc_gather(xa, xb, tab)
    return out[:, :D_MODEL].reshape(4096, 200, D_MODEL)


# speedup vs baseline: 1.0189x; 1.0001x over previous
import functools
import math

import jax
import jax.numpy as jnp
from jax import lax
from jax.experimental import pallas as pl
from jax.experimental.pallas import tpu as pltpu
from jax.experimental.pallas import tpu_sc as plsc

D_MODEL = 64
D_PAD = 128
SCALE = math.sqrt(D_MODEL)
VOCAB = 100000

NC, NS = 2, 16
NW = NC * NS

B_TOKENS = 4096 * 200
BPW = B_TOKENS // NW
SUB = 128
K = 2
CHUNK = SUB * K
NCHUNK = BPW // CHUNK
NROWS_W = BPW // SUB


def _scale_body(lut_ref, out_ref):
    v = lut_ref[...] * SCALE
    out_ref[...] = jnp.concatenate([v, jnp.zeros_like(v)], axis=1)


def _scaled_table(lut):
    return pl.pallas_call(
        _scale_body,
        out_shape=jax.ShapeDtypeStruct((VOCAB, D_PAD), jnp.float32),
        grid=(10,),
        in_specs=[pl.BlockSpec((VOCAB // 10, D_MODEL), lambda i: (i, 0))],
        out_specs=pl.BlockSpec((VOCAB // 10, D_PAD), lambda i: (i, 0)),
    )(lut)


NSEQ, TSEQ = 4096, 200
SEQ_W = NSEQ // NW
SPC = 1
CHUNK_T = SPC * TSEQ
NCHUNK_S = SEQ_W // SPC


@functools.partial(
    pl.kernel,
    out_type=jax.ShapeDtypeStruct((B_TOKENS, D_PAD), jnp.float32),
    mesh=plsc.VectorSubcoreMesh(core_axis_name="c", subcore_axis_name="s"),
    scratch_types=[
        pltpu.VMEM((SEQ_W, 128), jnp.int32),
        pltpu.VMEM((SEQ_W, 128), jnp.int32),
        pltpu.VMEM((3, CHUNK_T, D_PAD), jnp.float32),
        pltpu.SemaphoreType.DMA,
        pltpu.SemaphoreType.DMA,
        pltpu.SemaphoreType.DMA,
        pltpu.SemaphoreType.DMA,
        pltpu.SemaphoreType.DMA,
        pltpu.SemaphoreType.DMA,
    ],
    compiler_params=pltpu.CompilerParams(use_tc_tiling_on_sc=False),
)
def _sc_gather(xa_hbm, xb_hbm, tab_hbm, out_hbm, idx_a, idx_b, rows_v,
               sem_g0, sem_g1, sem_g2, sem_o0, sem_o1, sem_o2):
    wid = lax.axis_index("s") * NC + lax.axis_index("c")
    seq_base = wid * SEQ_W
    sem_g = (sem_g0, sem_g1, sem_g2)
    sem_o = (sem_o0, sem_o1, sem_o2)

    pltpu.sync_copy(xa_hbm.at[pl.ds(seq_base, SEQ_W)], idx_a)
    pltpu.sync_copy(xb_hbm.at[pl.ds(seq_base, SEQ_W)], idx_b)

    def fire_gathers(ii, b):
        for s in range(SPC):
            seq = ii * SPC + s
            pltpu.async_copy(
                tab_hbm.at[idx_a.at[seq]],
                rows_v.at[b].at[pl.ds(s * TSEQ, 128)],
                sem_g[b],
            )
            pltpu.async_copy(
                tab_hbm.at[idx_b.at[seq, pl.ds(0, TSEQ - 128)]],
                rows_v.at[b].at[pl.ds(s * TSEQ + 128, TSEQ - 128)],
                sem_g[b],
            )

    def out_slice(ii):
        return out_hbm.at[pl.ds((seq_base + ii * SPC) * TSEQ, CHUNK_T)]

    fire_gathers(0, 0)
    fire_gathers(1, 1)

    @pl.loop(0, NCHUNK_S, step=3)
    def _chunk(i):
        for b in range(3):
            ii = i + b
            b2 = (b + 2) % 3

            @pl.when(ii < NCHUNK_S)
            def _body():
                @pl.when(jnp.logical_and(ii >= 1, ii + 2 < NCHUNK_S))
                def _drain_out():
                    pltpu.make_async_copy(rows_v.at[b2], out_slice(ii), sem_o[b2]).wait()

                @pl.when(ii + 2 < NCHUNK_S)
                def _next_gathers():
                    fire_gathers(ii + 2, b2)

                pltpu.make_async_copy(tab_hbm.at[pl.ds(0, CHUNK_T)], rows_v.at[b], sem_g[b]).wait()
                pltpu.async_copy(rows_v.at[b], out_slice(ii), sem_o[b])

    for k in range(3):
        ii = NCHUNK_S - 3 + k
        pltpu.make_async_copy(rows_v.at[ii % 3], out_slice(ii), sem_o[ii % 3]).wait()


def kernel(x, lut):
    xi = x.astype(jnp.int32)
    xa = xi[:, :128]
    xb = jnp.pad(xi[:, 128:], ((0, 0), (0, 256 - TSEQ)))
    tab = _scaled_table(lut)
    out = _sc_gather(xa, xb, tab)
    return out[:, :D_MODEL].reshape(4096, 200, D_MODEL)
